# baseline (speedup 1.0000x reference)
import jax
import jax.numpy as jnp
from jax.experimental import pallas as pl
from jax.experimental.pallas import tpu as pltpu

_POOL = 1024
_K = 8
_D = 768
_B = 256
_S = 196
_BLK = 16
_GRID = _B // _BLK


_CB = 8
_NCHUNK = _B // _CB
_NBUF = 4


def _stream_body(x_any, pe_ref, xsum_ref, bufs, insems, outsems):
    def in_copy(c, buf):
        return pltpu.make_async_copy(
            x_any.at[pl.ds(c * _CB, _CB)], bufs.at[buf],
            insems.at[buf])

    def out_copy(c, buf):
        return pltpu.make_async_copy(
            bufs.at[buf],
            pe_ref.at[pl.ds(c * _CB, _CB), pl.ds(_K, _S), :],
            outsems.at[buf])

    for b in range(_NBUF - 1):
        in_copy(b, b).start()
    for i in range(_NCHUNK):
        if i + _NBUF - 1 < _NCHUNK:
            if i >= 1:
                out_copy(i - 1, (i - 1) % _NBUF).wait()
            in_copy(i + _NBUF - 1, (i + _NBUF - 1) % _NBUF).start()
        in_copy(i, i % _NBUF).wait()
        xsum_ref[pl.ds(i * _CB, _CB), :] = jnp.sum(bufs[i % _NBUF], axis=1)
        out_copy(i, i % _NBUF).start()
    for c in range(_NCHUNK - _NBUF, _NCHUNK):
        out_copy(c, c % _NBUF).wait()


def _head_body(xsum_ref, p_ref, sim_ref, idx_ref, rs_ref):
    xm = xsum_ref[...] * (1.0 / _S)
    xn = xm * jax.lax.rsqrt(jnp.maximum(
        jnp.sum(xm * xm, axis=1, keepdims=True), 1e-12))
    p = p_ref[...]
    pn = p * jax.lax.rsqrt(jnp.maximum(
        jnp.sum(p * p, axis=1, keepdims=True), 1e-12))
    sim = jax.lax.dot_general(
        xn, pn, (((1,), (1,)), ((), ())),
        preferred_element_type=jnp.float32)
    sim_ref[...] = sim

    iota = jax.lax.broadcasted_iota(jnp.int32, (_B, _POOL), 1)
    w = sim
    cols = []
    vsum = jnp.float32(0.0)
    for _ in range(_K):
        m = jnp.max(w, axis=1, keepdims=True)
        amax = jnp.min(jnp.where(w == m, iota, _POOL), axis=1,
                       keepdims=True)
        cols.append(amax)
        vsum = vsum + jnp.sum(m)
        w = jnp.where(iota == amax, -jnp.inf, w)
    idx_ref[...] = jnp.concatenate(cols, axis=1)
    rs_ref[0, 0] = vsum * (1.0 / _B)


def _gather_body(idx_ref, p_ref, pe_in_ref, pe_ref, rows_ref, sem):
    def body(r, _):
        b = r // _K
        k = r % _K
        v = idx_ref[b, k]
        rows_ref[b, pl.ds(k, 1), :] = p_ref[pl.ds(v, 1), :]
        return 0

    jax.lax.fori_loop(0, _B * _K, body, 0, unroll=8)
    cp = pltpu.make_async_copy(
        rows_ref, pe_ref.at[:, pl.ds(0, _K), :], sem)
    cp.start()
    cp.wait()


from jax import lax
from jax.experimental.pallas import tpu_sc as plsc

_NC = 2
_NS = 16
_NW = _NC * _NS
_BPW = _B // _NW
_RC = 49
_HC = _S // _RC


def _sc_copy_body(x_hbm, pe_hbm, bufs, insems, outsems):
    wid = lax.axis_index("s") * _NC + lax.axis_index("c")
    b0 = wid * _BPW

    chunks = [(j, h) for j in range(_BPW) for h in range(_HC)]

    def in_copy(c, buf):
        j, h = chunks[c]
        return pltpu.make_async_copy(
            x_hbm.at[b0 + j, pl.ds(h * _RC, _RC), :],
            bufs.at[buf], insems.at[buf])

    def out_copy(c, buf):
        j, h = chunks[c]
        return pltpu.make_async_copy(
            bufs.at[buf],
            pe_hbm.at[b0 + j, pl.ds(_K + h * _RC, _RC), :],
            outsems.at[buf])

    n = len(chunks)
    in_copy(0, 0).start()
    for i in range(n):
        cur = i % 2
        nxt = 1 - cur
        if i + 1 < n:
            if i >= 1:
                out_copy(i - 1, nxt).wait()
            in_copy(i + 1, nxt).start()
        in_copy(i, cur).wait()
        out_copy(i, cur).start()
    out_copy(n - 2, n % 2).wait()
    out_copy(n - 1, (n - 1) % 2).wait()


def _sc_copy_call(x_embed):
    mesh = plsc.VectorSubcoreMesh(core_axis_name="c", subcore_axis_name="s")
    f = pl.kernel(
        _sc_copy_body,
        out_type=jax.ShapeDtypeStruct((_B, _K + _S, _D), jnp.float32),
        mesh=mesh,
        compiler_params=pltpu.CompilerParams(use_tc_tiling_on_sc=False),
        scratch_types=[
            pltpu.VMEM((2, _RC, _D), jnp.float32),
            pltpu.SemaphoreType.DMA((2,)),
            pltpu.SemaphoreType.DMA((2,)),
        ],
    )
    return f(x_embed)


_MBLK = 16


def _mean_body(x_ref, xsum_ref):
    xsum_ref[...] = jnp.sum(x_ref[...], axis=1)


def _tc_mean(x_embed):
    (xsum,) = pl.pallas_call(
        _mean_body,
        grid=(_B // _MBLK,),
        in_specs=[pl.BlockSpec((_MBLK, _S, _D), lambda i: (i, 0, 0))],
        out_specs=[pl.BlockSpec((_MBLK, _D), lambda i: (i, 0))],
        out_shape=[jax.ShapeDtypeStruct((_B, _D), jnp.float32)],
    )(x_embed)
    return xsum


def kernel(x_embed, prompt):
    pe = _sc_copy_call(x_embed)
    xsum = _tc_mean(x_embed)
    sim = jnp.zeros((_B, _POOL), jnp.float32)
    idx = jnp.zeros((_B, _K), jnp.int32)
    rs = jnp.sum(xsum)
    return pe, sim, rs, idx


# speedup vs baseline: 1.4829x; 1.4829x over previous
import jax
import jax.numpy as jnp
from jax import lax
from jax.experimental import pallas as pl
from jax.experimental.pallas import tpu as pltpu
from jax.experimental.pallas import tpu_sc as plsc

_POOL = 1024
_K = 8
_D = 768
_B = 256
_S = 196

_SC_NC = 2
_SC_NS = 16
_SC_NW = _SC_NC * _SC_NS
_SC_BPW = _B // _SC_NW
_SC_CC = 256
_SC_NCC = _D // _SC_CC
_LANES = 16


def _sc_mean_body(x_hbm, xsum_hbm, bufs, acc_ref, insems, outsem):
    wid = lax.axis_index("s") * _SC_NC + lax.axis_index("c")
    b0 = wid * _SC_BPW
    chunks = [(j, c) for j in range(_SC_BPW) for c in range(_SC_NCC)]
    n = len(chunks)

    def in_copy(i, slot):
        j, c = chunks[i]
        return pltpu.make_async_copy(
            x_hbm.at[b0 + j, :, pl.ds(c * _SC_CC, _SC_CC)],
            bufs.at[slot], insems.at[slot])

    in_copy(0, 0).start()
    nv = _SC_CC // _LANES
    for i in range(n):
        cur = i % 2
        if i + 1 < n:
            in_copy(i + 1, 1 - cur).start()
        in_copy(i, cur).wait()
        j, c = chunks[i]

        def row_body(r, accs):
            return tuple(
                accs[v] + bufs[cur, r, pl.ds(v * _LANES, _LANES)]
                for v in range(nv))

        accs = tuple(jnp.zeros((_LANES,), jnp.float32) for _ in range(nv))
        accs = jax.lax.fori_loop(0, _S, row_body, accs, unroll=4)
        for v in range(nv):
            acc_ref[j, pl.ds(c * _SC_CC + v * _LANES, _LANES)] = accs[v]

    cp = pltpu.make_async_copy(
        acc_ref, xsum_hbm.at[pl.ds(b0, _SC_BPW), :], outsem)
    cp.start()
    cp.wait()


def _sc_mean(x_embed):
    mesh = plsc.VectorSubcoreMesh(core_axis_name="c", subcore_axis_name="s")
    f = pl.kernel(
        _sc_mean_body,
        out_type=jax.ShapeDtypeStruct((_B, _D), jnp.float32),
        mesh=mesh,
        scratch_types=[
            pltpu.VMEM((2, _S, _SC_CC), jnp.float32),
            pltpu.VMEM((_SC_BPW, _D), jnp.float32),
            pltpu.SemaphoreType.DMA((2,)),
            pltpu.SemaphoreType.DMA,
        ],
    )
    return f(x_embed)


def _head_body(xsum_ref, p_ref, sim_ref, idx_ref, rs_ref):
    xm = xsum_ref[...] * (1.0 / _S)
    xn = xm * jax.lax.rsqrt(jnp.maximum(
        jnp.sum(xm * xm, axis=1, keepdims=True), 1e-12))
    p = p_ref[...]
    pn = p * jax.lax.rsqrt(jnp.maximum(
        jnp.sum(p * p, axis=1, keepdims=True), 1e-12))
    sim = jax.lax.dot_general(
        xn, pn, (((1,), (1,)), ((), ())),
        preferred_element_type=jnp.float32)
    sim_ref[...] = sim

    iota = jax.lax.broadcasted_iota(jnp.int32, (_B, _POOL), 1)
    w = sim
    cols = []
    vsum = jnp.float32(0.0)
    for _ in range(_K):
        m = jnp.max(w, axis=1, keepdims=True)
        amax = jnp.min(jnp.where(w == m, iota, _POOL), axis=1,
                       keepdims=True)
        cols.append(amax)
        vsum = vsum + jnp.sum(m)
        w = jnp.where(iota == amax, -jnp.inf, w)
    idx_ref[...] = jnp.concatenate(cols, axis=1)
    rs_ref[0, 0] = vsum * (1.0 / _B)


def _gather_body(idx_ref, p_ref, pe_in_ref, pe_ref, rows_ref, sem):
    def body(r, _):
        b = r // _K
        k = r % _K
        v = idx_ref[b, k]
        rows_ref[b, pl.ds(k, 1), :] = p_ref[pl.ds(v, 1), :]
        return 0

    jax.lax.fori_loop(0, _B * _K, body, 0, unroll=8)
    cp = pltpu.make_async_copy(
        rows_ref, pe_ref.at[:, pl.ds(0, _K), :], sem)
    cp.start()
    cp.wait()


def kernel(x_embed, prompt):
    xsum = _sc_mean(x_embed)

    pe_partial = jnp.concatenate(
        [jnp.zeros((_B, _K, _D), jnp.float32), x_embed], axis=1)

    sim, idx, rs = pl.pallas_call(
        _head_body,
        in_specs=[
            pl.BlockSpec((_B, _D), lambda: (0, 0)),
            pl.BlockSpec((_POOL, _D), lambda: (0, 0)),
        ],
        out_specs=[
            pl.BlockSpec((_B, _POOL), lambda: (0, 0)),
            pl.BlockSpec((_B, _K), lambda: (0, 0)),
            pl.BlockSpec(block_shape=(1, 1), index_map=lambda: (0, 0),
                         memory_space=pltpu.SMEM),
        ],
        out_shape=[
            jax.ShapeDtypeStruct((_B, _POOL), jnp.float32),
            jax.ShapeDtypeStruct((_B, _K), jnp.int32),
            jax.ShapeDtypeStruct((1, 1), jnp.float32),
        ],
    )(xsum, prompt)

    pe = pl.pallas_call(
        _gather_body,
        in_specs=[
            pl.BlockSpec(memory_space=pltpu.SMEM),
            pl.BlockSpec((_POOL, _D), lambda: (0, 0)),
            pl.BlockSpec(memory_space=pl.ANY),
        ],
        out_specs=pl.BlockSpec(memory_space=pl.ANY),
        out_shape=jax.ShapeDtypeStruct((_B, _K + _S, _D), jnp.float32),
        scratch_shapes=[pltpu.VMEM((_B, _K, _D), jnp.float32),
                        pltpu.SemaphoreType.DMA],
        input_output_aliases={2: 0},
    )(idx, prompt, pe_partial)

    return pe, sim, rs.reshape(()), idx


# speedup vs baseline: 1.4893x; 1.0043x over previous
import jax
import jax.numpy as jnp
from jax import lax
from jax.experimental import pallas as pl
from jax.experimental.pallas import tpu as pltpu
from jax.experimental.pallas import tpu_sc as plsc

_POOL = 1024
_K = 8
_D = 768
_B = 256
_S = 196

_SC_NC = 2
_SC_NS = 16
_SC_NW = _SC_NC * _SC_NS
_SC_BPW = _B // _SC_NW
_SC_CC = 128
_SC_NCC = _D // _SC_CC
_LANES = 16


def _sc_mean_body(x_hbm, xsum_hbm, bufs, acc_ref, insems, outsem):
    wid = lax.axis_index("s") * _SC_NC + lax.axis_index("c")
    b0 = wid * _SC_BPW
    chunks = [(j, c) for j in range(_SC_BPW) for c in range(_SC_NCC)]
    n = len(chunks)

    def in_copy(i, slot):
        j, c = chunks[i]
        return pltpu.make_async_copy(
            x_hbm.at[b0 + j, :, pl.ds(c * _SC_CC, _SC_CC)],
            bufs.at[slot], insems.at[slot])

    _NB = 4
    for s in range(_NB - 1):
        in_copy(s, s).start()
    nv = _SC_CC // _LANES
    for i in range(n):
        cur = i % _NB
        if i + _NB - 1 < n:
            in_copy(i + _NB - 1, (i + _NB - 1) % _NB).start()
        in_copy(i, cur).wait()
        j, c = chunks[i]

        def row_body(r, accs):
            return tuple(
                accs[v] + bufs[cur, r, pl.ds(v * _LANES, _LANES)]
                for v in range(nv))

        accs = tuple(jnp.zeros((_LANES,), jnp.float32) for _ in range(nv))
        accs = jax.lax.fori_loop(0, _S, row_body, accs, unroll=4)
        for v in range(nv):
            acc_ref[j, pl.ds(c * _SC_CC + v * _LANES, _LANES)] = accs[v]

    cp = pltpu.make_async_copy(
        acc_ref, xsum_hbm.at[pl.ds(b0, _SC_BPW), :], outsem)
    cp.start()
    cp.wait()


def _sc_mean(x_embed):
    mesh = plsc.VectorSubcoreMesh(core_axis_name="c", subcore_axis_name="s")
    f = pl.kernel(
        _sc_mean_body,
        out_type=jax.ShapeDtypeStruct((_B, _D), jnp.float32),
        mesh=mesh,
        scratch_types=[
            pltpu.VMEM((4, _S, _SC_CC), jnp.float32),
            pltpu.VMEM((_SC_BPW, _D), jnp.float32),
            pltpu.SemaphoreType.DMA((4,)),
            pltpu.SemaphoreType.DMA,
        ],
    )
    return f(x_embed)


def _head_body(xsum_ref, p_ref, sim_ref, idx_ref, rs_ref):
    xm = xsum_ref[...] * (1.0 / _S)
    xn = xm * jax.lax.rsqrt(jnp.maximum(
        jnp.sum(xm * xm, axis=1, keepdims=True), 1e-12))
    p = p_ref[...]
    pn = p * jax.lax.rsqrt(jnp.maximum(
        jnp.sum(p * p, axis=1, keepdims=True), 1e-12))
    sim = jax.lax.dot_general(
        xn, pn, (((1,), (1,)), ((), ())),
        preferred_element_type=jnp.float32)
    sim_ref[...] = sim

    iota = jax.lax.broadcasted_iota(jnp.int32, (_B, _POOL), 1)
    w = sim
    cols = []
    vsum = jnp.float32(0.0)
    for _ in range(_K):
        m = jnp.max(w, axis=1, keepdims=True)
        amax = jnp.min(jnp.where(w == m, iota, _POOL), axis=1,
                       keepdims=True)
        cols.append(amax)
        vsum = vsum + jnp.sum(m)
        w = jnp.where(iota == amax, -jnp.inf, w)
    idx_ref[...] = jnp.concatenate(cols, axis=1)
    rs_ref[0, 0] = vsum * (1.0 / _B)


def _gather_body(idx_ref, p_ref, pe_in_ref, pe_ref, rows_ref, sem):
    def body(r, _):
        b = r // _K
        k = r % _K
        v = idx_ref[b, k]
        rows_ref[b, pl.ds(k, 1), :] = p_ref[pl.ds(v, 1), :]
        return 0

    jax.lax.fori_loop(0, _B * _K, body, 0, unroll=8)
    cp = pltpu.make_async_copy(
        rows_ref, pe_ref.at[:, pl.ds(0, _K), :], sem)
    cp.start()
    cp.wait()


def kernel(x_embed, prompt):
    xsum = _sc_mean(x_embed)

    pe_partial = jnp.concatenate(
        [jnp.zeros((_B, _K, _D), jnp.float32), x_embed], axis=1)

    sim, idx, rs = pl.pallas_call(
        _head_body,
        in_specs=[
            pl.BlockSpec((_B, _D), lambda: (0, 0)),
            pl.BlockSpec((_POOL, _D), lambda: (0, 0)),
        ],
        out_specs=[
            pl.BlockSpec((_B, _POOL), lambda: (0, 0)),
            pl.BlockSpec((_B, _K), lambda: (0, 0)),
            pl.BlockSpec(block_shape=(1, 1), index_map=lambda: (0, 0),
                         memory_space=pltpu.SMEM),
        ],
        out_shape=[
            jax.ShapeDtypeStruct((_B, _POOL), jnp.float32),
            jax.ShapeDtypeStruct((_B, _K), jnp.int32),
            jax.ShapeDtypeStruct((1, 1), jnp.float32),
        ],
    )(xsum, prompt)

    pe = pl.pallas_call(
        _gather_body,
        in_specs=[
            pl.BlockSpec(memory_space=pltpu.SMEM),
            pl.BlockSpec((_POOL, _D), lambda: (0, 0)),
            pl.BlockSpec(memory_space=pl.ANY),
        ],
        out_specs=pl.BlockSpec(memory_space=pl.ANY),
        out_shape=jax.ShapeDtypeStruct((_B, _K + _S, _D), jnp.float32),
        scratch_shapes=[pltpu.VMEM((_B, _K, _D), jnp.float32),
                        pltpu.SemaphoreType.DMA],
        input_output_aliases={2: 0},
    )(idx, prompt, pe_partial)

    return pe, sim, rs.reshape(()), idx


# speedup vs baseline: 1.5250x; 1.0240x over previous
import jax
import jax.numpy as jnp
from jax import lax
from jax.experimental import pallas as pl
from jax.experimental.pallas import tpu as pltpu
from jax.experimental.pallas import tpu_sc as plsc

_POOL = 1024
_K = 8
_D = 768
_B = 256
_S = 196

_SC_NC = 2
_SC_NS = 16
_SC_NW = _SC_NC * _SC_NS
_SC_B0 = 64
_SC_BPW = (_B - _SC_B0) // _SC_NW
_SC_CC = 128
_SC_NCC = _D // _SC_CC
_LANES = 16


def _sc_mean_body(x_hbm, xsum_hbm, bufs, acc_ref, insems, outsem):
    wid = lax.axis_index("s") * _SC_NC + lax.axis_index("c")
    b0 = _SC_B0 + wid * _SC_BPW
    chunks = [(j, c) for j in range(_SC_BPW) for c in range(_SC_NCC)]
    n = len(chunks)

    def in_copy(i, slot):
        j, c = chunks[i]
        return pltpu.make_async_copy(
            x_hbm.at[b0 + j, :, pl.ds(c * _SC_CC, _SC_CC)],
            bufs.at[slot], insems.at[slot])

    _NB = 3
    for s in range(_NB - 1):
        in_copy(s, s).start()
    nv = _SC_CC // _LANES
    for i in range(n):
        cur = i % _NB
        if i + _NB - 1 < n:
            in_copy(i + _NB - 1, (i + _NB - 1) % _NB).start()
        in_copy(i, cur).wait()
        j, c = chunks[i]

        def row_body(r, accs):
            return tuple(
                accs[v] + bufs[cur, r, pl.ds(v * _LANES, _LANES)]
                for v in range(nv))

        accs = tuple(jnp.zeros((_LANES,), jnp.float32) for _ in range(nv))
        accs = jax.lax.fori_loop(0, _S, row_body, accs, unroll=4)
        for v in range(nv):
            acc_ref[j, 0, pl.ds(c * _SC_CC + v * _LANES, _LANES)] = accs[v]

    cp = pltpu.make_async_copy(
        acc_ref, xsum_hbm.at[pl.ds(wid * _SC_BPW, _SC_BPW)], outsem)
    cp.start()
    cp.wait()


def _sc_mean(x_embed):
    mesh = plsc.VectorSubcoreMesh(core_axis_name="c", subcore_axis_name="s")
    f = pl.kernel(
        _sc_mean_body,
        out_type=jax.ShapeDtypeStruct((_B - _SC_B0, 8, _D), jnp.float32),
        mesh=mesh,
        scratch_types=[
            pltpu.VMEM((3, _S, _SC_CC), jnp.float32),
            pltpu.VMEM((_SC_BPW, 8, _D), jnp.float32),
            pltpu.SemaphoreType.DMA((3,)),
            pltpu.SemaphoreType.DMA,
        ],
    )
    return f(x_embed)


_TCBLK = 16


def _tc_mean_body(x_ref, xsum_ref):
    xsum_ref[...] = jnp.sum(x_ref[...], axis=1)


def _head_body(xsuma_ref, xsumb_ref, p_ref, sim_ref, idx_ref, rs_ref):
    xsum = jnp.concatenate(
        [xsuma_ref[...], xsumb_ref[:, 0, :]], axis=0)
    xm = xsum * (1.0 / _S)
    xn = xm * jax.lax.rsqrt(jnp.maximum(
        jnp.sum(xm * xm, axis=1, keepdims=True), 1e-12))
    p = p_ref[...]
    pn = p * jax.lax.rsqrt(jnp.maximum(
        jnp.sum(p * p, axis=1, keepdims=True), 1e-12))
    sim = jax.lax.dot_general(
        xn, pn, (((1,), (1,)), ((), ())),
        preferred_element_type=jnp.float32)
    sim_ref[...] = sim

    iota = jax.lax.broadcasted_iota(jnp.int32, (_B, _POOL), 1)
    w = sim
    cols = []
    vsum = jnp.float32(0.0)
    for _ in range(_K):
        m = jnp.max(w, axis=1, keepdims=True)
        amax = jnp.min(jnp.where(w == m, iota, _POOL), axis=1,
                       keepdims=True)
        cols.append(amax)
        vsum = vsum + jnp.sum(m)
        w = jnp.where(iota == amax, -jnp.inf, w)
    idx_ref[...] = jnp.concatenate(cols, axis=1)
    rs_ref[0, 0] = vsum * (1.0 / _B)


def _gather_body(idx_ref, p_ref, pe_in_ref, pe_ref, rows_ref, sem):
    def body(r, _):
        b = r // _K
        k = r % _K
        v = idx_ref[b, k]
        rows_ref[b, pl.ds(k, 1), :] = p_ref[pl.ds(v, 1), :]
        return 0

    jax.lax.fori_loop(0, _B * _K, body, 0, unroll=8)
    cp = pltpu.make_async_copy(
        rows_ref, pe_ref.at[:, pl.ds(0, _K), :], sem)
    cp.start()
    cp.wait()


def kernel(x_embed, prompt):
    xsumb = _sc_mean(x_embed)

    pe_partial = jnp.concatenate(
        [jnp.zeros((_B, _K, _D), jnp.float32), x_embed], axis=1)

    (xsuma,) = pl.pallas_call(
        _tc_mean_body,
        grid=(_SC_B0 // _TCBLK,),
        in_specs=[pl.BlockSpec((_TCBLK, _S, _D), lambda i: (i, 0, 0))],
        out_specs=[pl.BlockSpec((_TCBLK, _D), lambda i: (i, 0))],
        out_shape=[jax.ShapeDtypeStruct((_SC_B0, _D), jnp.float32)],
    )(x_embed)

    sim, idx, rs = pl.pallas_call(
        _head_body,
        in_specs=[
            pl.BlockSpec((_SC_B0, _D), lambda: (0, 0)),
            pl.BlockSpec((_B - _SC_B0, 8, _D), lambda: (0, 0, 0)),
            pl.BlockSpec((_POOL, _D), lambda: (0, 0)),
        ],
        out_specs=[
            pl.BlockSpec((_B, _POOL), lambda: (0, 0)),
            pl.BlockSpec((_B, _K), lambda: (0, 0)),
            pl.BlockSpec(block_shape=(1, 1), index_map=lambda: (0, 0),
                         memory_space=pltpu.SMEM),
        ],
        out_shape=[
            jax.ShapeDtypeStruct((_B, _POOL), jnp.float32),
            jax.ShapeDtypeStruct((_B, _K), jnp.int32),
            jax.ShapeDtypeStruct((1, 1), jnp.float32),
        ],
    )(xsuma, xsumb, prompt)

    pe = pl.pallas_call(
        _gather_body,
        in_specs=[
            pl.BlockSpec(memory_space=pltpu.SMEM),
            pl.BlockSpec((_POOL, _D), lambda: (0, 0)),
            pl.BlockSpec(memory_space=pl.ANY),
        ],
        out_specs=pl.BlockSpec(memory_space=pl.ANY),
        out_shape=jax.ShapeDtypeStruct((_B, _K + _S, _D), jnp.float32),
        scratch_shapes=[pltpu.VMEM((_B, _K, _D), jnp.float32),
                        pltpu.SemaphoreType.DMA],
        input_output_aliases={2: 0},
    )(idx, prompt, pe_partial)

    return pe, sim, rs.reshape(()), idx


# speedup vs baseline: 1.7705x; 1.1609x over previous
import jax
import jax.numpy as jnp
from jax import lax
from jax.experimental import pallas as pl
from jax.experimental.pallas import tpu as pltpu
from jax.experimental.pallas import tpu_sc as plsc

_POOL = 1024
_K = 8
_D = 768
_B = 256
_S = 196
_BLK = 16
_GRID = _B // _BLK


_CB = 8
_NCHUNK = _B // _CB
_NBUF = 4


def _stream_body(x_any, pe_ref, xsum_ref, bufs, insems, outsems):
    def in_copy(c, buf):
        return pltpu.make_async_copy(
            x_any.at[pl.ds(c * _CB, _CB)], bufs.at[buf],
            insems.at[buf])

    def out_copy(c, buf):
        return pltpu.make_async_copy(
            bufs.at[buf],
            pe_ref.at[pl.ds(c * _CB, _CB), pl.ds(_K, _S), :],
            outsems.at[buf])

    for b in range(_NBUF - 1):
        in_copy(b, b).start()
    for i in range(_NCHUNK):
        if i + _NBUF - 1 < _NCHUNK:
            if i >= 1:
                out_copy(i - 1, (i - 1) % _NBUF).wait()
            in_copy(i + _NBUF - 1, (i + _NBUF - 1) % _NBUF).start()
        in_copy(i, i % _NBUF).wait()
        xsum_ref[pl.ds(i * _CB, _CB), :] = jnp.sum(bufs[i % _NBUF], axis=1)
        out_copy(i, i % _NBUF).start()
    for c in range(_NCHUNK - _NBUF, _NCHUNK):
        out_copy(c, c % _NBUF).wait()


def _head_body(xsum_ref, p_ref, sim_ref, idx_ref, rs_ref):
    xm = xsum_ref[...] * (1.0 / _S)
    xn = xm * jax.lax.rsqrt(jnp.maximum(
        jnp.sum(xm * xm, axis=1, keepdims=True), 1e-12))
    p = p_ref[...]
    pn = p * jax.lax.rsqrt(jnp.maximum(
        jnp.sum(p * p, axis=1, keepdims=True), 1e-12))
    sim = jax.lax.dot_general(
        xn, pn, (((1,), (1,)), ((), ())),
        preferred_element_type=jnp.float32)
    sim_ref[...] = sim

    iota = jax.lax.broadcasted_iota(jnp.int32, (_B, _POOL), 1)
    w = sim
    cols = []
    vsum = jnp.float32(0.0)
    for _ in range(_K):
        m = jnp.max(w, axis=1, keepdims=True)
        amax = jnp.min(jnp.where(w == m, iota, _POOL), axis=1,
                       keepdims=True)
        cols.append(amax)
        vsum = vsum + jnp.sum(m)
        w = jnp.where(iota == amax, -jnp.inf, w)
    idx_ref[...] = jnp.concatenate(cols, axis=1)
    rs_ref[0, 0] = vsum * (1.0 / _B)


_SC_NC = 2
_SC_BPW = _B // 32


def _sc_gather_body(idx_hbm, p_hbm, pe_hbm, idxbuf, rows, isem, gsems, wsems):
    wid = lax.axis_index("s") * _SC_NC + lax.axis_index("c")
    b0 = wid * _SC_BPW

    cp = pltpu.make_async_copy(
        idx_hbm.at[pl.ds(b0, _SC_BPW), :], idxbuf, isem)
    cp.start()
    cp.wait()

    def gather_copy(j, slot):
        return pltpu.make_async_copy(
            p_hbm.at[idxbuf.at[j]], rows.at[slot], gsems.at[slot])

    def write_copy(j, slot):
        return pltpu.make_async_copy(
            rows.at[slot], pe_hbm.at[b0 + j, pl.ds(0, _K), :],
            wsems.at[slot])

    for j in range(_SC_BPW):
        slot = j % 2
        if j >= 2:
            write_copy(j - 2, slot).wait()
        g = gather_copy(j, slot)
        g.start()
        g.wait()
        write_copy(j, slot).start()
    write_copy(_SC_BPW - 2, 0).wait()
    write_copy(_SC_BPW - 1, 1).wait()


def _sc_gather(idx, prompt, pe_ref):
    mesh = plsc.VectorSubcoreMesh(core_axis_name="c", subcore_axis_name="s")
    f = pl.kernel(
        _sc_gather_body,
        out_type=(),
        mesh=mesh,
        scratch_types=[
            pltpu.VMEM((_SC_BPW, _K), jnp.int32),
            pltpu.VMEM((2, _K, _D), jnp.float32),
            pltpu.SemaphoreType.DMA,
            pltpu.SemaphoreType.DMA((2,)),
            pltpu.SemaphoreType.DMA((2,)),
        ],
    )
    f(idx, prompt, pe_ref)


def kernel(x_embed, prompt):
    pe_partial, xsum = pl.pallas_call(
        _stream_body,
        in_specs=[pl.BlockSpec(memory_space=pl.ANY)],
        out_specs=[
            pl.BlockSpec(memory_space=pl.ANY),
            pl.BlockSpec((_B, _D), lambda: (0, 0)),
        ],
        out_shape=[
            jax.ShapeDtypeStruct((_B, _K + _S, _D), jnp.float32),
            jax.ShapeDtypeStruct((_B, _D), jnp.float32),
        ],
        scratch_shapes=[
            pltpu.VMEM((_NBUF, _CB, _S, _D), jnp.float32),
            pltpu.SemaphoreType.DMA((_NBUF,)),
            pltpu.SemaphoreType.DMA((_NBUF,)),
        ],
    )(x_embed)

    sim, idx, rs = pl.pallas_call(
        _head_body,
        in_specs=[
            pl.BlockSpec((_B, _D), lambda: (0, 0)),
            pl.BlockSpec((_POOL, _D), lambda: (0, 0)),
        ],
        out_specs=[
            pl.BlockSpec((_B, _POOL), lambda: (0, 0)),
            pl.BlockSpec((_B, _K), lambda: (0, 0)),
            pl.BlockSpec(block_shape=(1, 1), index_map=lambda: (0, 0),
                         memory_space=pltpu.SMEM),
        ],
        out_shape=[
            jax.ShapeDtypeStruct((_B, _POOL), jnp.float32),
            jax.ShapeDtypeStruct((_B, _K), jnp.int32),
            jax.ShapeDtypeStruct((1, 1), jnp.float32),
        ],
    )(xsum, prompt)

    pe_ref = jax.new_ref(pe_partial)
    _sc_gather(idx, prompt, pe_ref)
    pe = pe_ref[...]

    return pe, sim, rs.reshape(()), idx


# speedup vs baseline: 1.7818x; 1.0064x over previous
import jax
import jax.numpy as jnp
from jax import lax
from jax.experimental import pallas as pl
from jax.experimental.pallas import tpu as pltpu
from jax.experimental.pallas import tpu_sc as plsc

_POOL = 1024
_K = 8
_D = 768
_B = 256
_S = 196
_BLK = 16
_GRID = _B // _BLK


_CB = 8
_NCHUNK = _B // _CB
_NBUF = 4


def _stream_body(x_any, pe_ref, xsum_ref, bufs, insems, outsems):
    def in_copy(c, buf):
        return pltpu.make_async_copy(
            x_any.at[pl.ds(c * _CB, _CB)], bufs.at[buf],
            insems.at[buf])

    def out_copy(c, buf):
        return pltpu.make_async_copy(
            bufs.at[buf],
            pe_ref.at[pl.ds(c * _CB, _CB), pl.ds(_K, _S), :],
            outsems.at[buf])

    for b in range(_NBUF - 1):
        in_copy(b, b).start()
    for i in range(_NCHUNK):
        if i + _NBUF - 1 < _NCHUNK:
            if i >= 1:
                out_copy(i - 1, (i - 1) % _NBUF).wait()
            in_copy(i + _NBUF - 1, (i + _NBUF - 1) % _NBUF).start()
        in_copy(i, i % _NBUF).wait()
        xsum_ref[pl.ds(i * _CB, _CB), :] = jnp.sum(bufs[i % _NBUF], axis=1)
        out_copy(i, i % _NBUF).start()
    for c in range(_NCHUNK - _NBUF, _NCHUNK):
        out_copy(c, c % _NBUF).wait()


def _head_body(xsum_ref, p_ref, sim_ref, idx_ref, rs_ref):
    xm = xsum_ref[...] * (1.0 / _S)
    xn = xm * jax.lax.rsqrt(jnp.maximum(
        jnp.sum(xm * xm, axis=1, keepdims=True), 1e-12))
    p = p_ref[...]
    pn = p * jax.lax.rsqrt(jnp.maximum(
        jnp.sum(p * p, axis=1, keepdims=True), 1e-12))
    sim = jax.lax.dot_general(
        xn, pn, (((1,), (1,)), ((), ())),
        preferred_element_type=jnp.float32)
    sim_ref[...] = sim

    iota = jax.lax.broadcasted_iota(jnp.int32, (_B, _POOL), 1)
    w = sim
    cols = []
    vsum = jnp.float32(0.0)
    for _ in range(_K):
        m = jnp.max(w, axis=1, keepdims=True)
        amax = jnp.min(jnp.where(w == m, iota, _POOL), axis=1,
                       keepdims=True)
        cols.append(amax)
        vsum = vsum + jnp.sum(m)
        w = jnp.where(iota == amax, -jnp.inf, w)
    idx_ref[...] = jnp.concatenate(cols, axis=1)
    rs_ref[0, 0] = vsum * (1.0 / _B)


_SC_NC = 2
_SC_BPW = _B // 32


def _sc_gather_body(idx_hbm, p_hbm, pe_hbm, idxbuf, rows, isem, gsem, wsems):
    wid = lax.axis_index("s") * _SC_NC + lax.axis_index("c")
    b0 = wid * _SC_BPW
    nrows = _SC_BPW * _K

    cp = pltpu.make_async_copy(
        idx_hbm.at[pl.ds(wid * nrows, nrows)], idxbuf, isem)
    cp.start()
    cp.wait()

    g = pltpu.make_async_copy(p_hbm.at[idxbuf], rows, gsem)
    g.start()
    g.wait()

    def write_copy(j):
        return pltpu.make_async_copy(
            rows.at[pl.ds(j * _K, _K)],
            pe_hbm.at[b0 + j, pl.ds(0, _K), :], wsems.at[j])

    for j in range(_SC_BPW):
        write_copy(j).start()
    for j in range(_SC_BPW):
        write_copy(j).wait()


def _sc_gather(idx, prompt, pe_ref):
    mesh = plsc.VectorSubcoreMesh(core_axis_name="c", subcore_axis_name="s")
    f = pl.kernel(
        _sc_gather_body,
        out_type=(),
        mesh=mesh,
        scratch_types=[
            pltpu.VMEM((_SC_BPW * _K,), jnp.int32),
            pltpu.VMEM((_SC_BPW * _K, _D), jnp.float32),
            pltpu.SemaphoreType.DMA,
            pltpu.SemaphoreType.DMA,
            pltpu.SemaphoreType.DMA((_SC_BPW,)),
        ],
    )
    f(idx.reshape(-1), prompt, pe_ref)


def kernel(x_embed, prompt):
    pe_partial, xsum = pl.pallas_call(
        _stream_body,
        in_specs=[pl.BlockSpec(memory_space=pl.ANY)],
        out_specs=[
            pl.BlockSpec(memory_space=pl.ANY),
            pl.BlockSpec((_B, _D), lambda: (0, 0)),
        ],
        out_shape=[
            jax.ShapeDtypeStruct((_B, _K + _S, _D), jnp.float32),
            jax.ShapeDtypeStruct((_B, _D), jnp.float32),
        ],
        scratch_shapes=[
            pltpu.VMEM((_NBUF, _CB, _S, _D), jnp.float32),
            pltpu.SemaphoreType.DMA((_NBUF,)),
            pltpu.SemaphoreType.DMA((_NBUF,)),
        ],
    )(x_embed)

    sim, idx, rs = pl.pallas_call(
        _head_body,
        in_specs=[
            pl.BlockSpec((_B, _D), lambda: (0, 0)),
            pl.BlockSpec((_POOL, _D), lambda: (0, 0)),
        ],
        out_specs=[
            pl.BlockSpec((_B, _POOL), lambda: (0, 0)),
            pl.BlockSpec((_B, _K), lambda: (0, 0)),
            pl.BlockSpec(block_shape=(1, 1), index_map=lambda: (0, 0),
                         memory_space=pltpu.SMEM),
        ],
        out_shape=[
            jax.ShapeDtypeStruct((_B, _POOL), jnp.float32),
            jax.ShapeDtypeStruct((_B, _K), jnp.int32),
            jax.ShapeDtypeStruct((1, 1), jnp.float32),
        ],
    )(xsum, prompt)

    pe_ref = jax.new_ref(pe_partial)
    _sc_gather(idx, prompt, pe_ref)
    pe = pe_ref[...]

    return pe, sim, rs.reshape(()), idx
